# speedup vs baseline: 1.5419x; 1.5419x over previous
import functools

import jax
import jax.numpy as jnp
from jax import lax
from jax.experimental import pallas as pl
from jax.experimental.pallas import tpu as pltpu
from jax.experimental.pallas import tpu_sc as plsc

_info = plsc.get_sparse_core_info()
_NC, _NS = _info.num_cores, _info.num_subcores
_NW = _NC * _NS

_CHUNK = 64


@functools.lru_cache(maxsize=None)
def _make_gather(B, V, D):
    assert B % (_NW * _CHUNK) == 0
    b_per_w = B // _NW
    n_chunks = b_per_w // _CHUNK
    mesh = plsc.VectorSubcoreMesh(core_axis_name="c", subcore_axis_name="s")

    @functools.partial(
        pl.kernel,
        out_type=jax.ShapeDtypeStruct((B, D), jnp.float32),
        mesh=mesh,
        scratch_types=[
            pltpu.VMEM((b_per_w,), jnp.int32),
            pltpu.VMEM((_CHUNK, D), jnp.float32),
            pltpu.SemaphoreType.DMA,
        ],
    )
    def gather_kernel(table_hbm, idx_hbm, out_hbm, idx_v, rows_v, sem):
        wid = lax.axis_index("s") * _NC + lax.axis_index("c")
        base = wid * b_per_w
        pltpu.sync_copy(idx_hbm.at[pl.ds(base, b_per_w)], idx_v)
        for g in range(n_chunks):
            pltpu.async_copy(
                table_hbm.at[idx_v.at[pl.ds(g * _CHUNK, _CHUNK)]], rows_v, sem
            ).wait()
            pltpu.sync_copy(rows_v, out_hbm.at[pl.ds(base + g * _CHUNK, _CHUNK)])

    return gather_kernel


def kernel(tokens, W_E):
    B = tokens.size
    V, D = W_E.shape
    idx = tokens.reshape(B).astype(jnp.int32)
    out = _make_gather(B, V, D)(W_E, idx)
    return out.reshape(*tokens.shape, D)


# speedup vs baseline: 1.5765x; 1.0225x over previous
import functools

import jax
import jax.numpy as jnp
from jax import lax
from jax.experimental import pallas as pl
from jax.experimental.pallas import tpu as pltpu
from jax.experimental.pallas import tpu_sc as plsc

_info = plsc.get_sparse_core_info()
_NC, _NS = _info.num_cores, _info.num_subcores
_NW = _NC * _NS

_CHUNK = 32
_NBUF = 3


@functools.lru_cache(maxsize=None)
def _make_gather(B, V, D):
    assert B % (_NW * _CHUNK) == 0
    b_per_w = B // _NW
    n_chunks = b_per_w // _CHUNK
    mesh = plsc.VectorSubcoreMesh(core_axis_name="c", subcore_axis_name="s")

    @functools.partial(
        pl.kernel,
        out_type=jax.ShapeDtypeStruct((B, D), jnp.float32),
        mesh=mesh,
        scratch_types=[
            pltpu.VMEM((b_per_w,), jnp.int32),
            pltpu.VMEM((_NBUF, _CHUNK, D), jnp.float32),
            pltpu.SemaphoreType.DMA,
        ]
        + [pltpu.SemaphoreType.DMA] * _NBUF,
    )
    def gather_kernel(table_hbm, idx_hbm, out_hbm, idx_v, rows_v, gsem, *wsems):
        wid = lax.axis_index("s") * _NC + lax.axis_index("c")
        base = wid * b_per_w
        pltpu.sync_copy(idx_hbm.at[pl.ds(base, b_per_w)], idx_v)

        def start_gather(g):
            return pltpu.async_copy(
                table_hbm.at[idx_v.at[pl.ds(g * _CHUNK, _CHUNK)]],
                rows_v.at[g % _NBUF],
                gsem,
            )

        gathers = [None] * n_chunks
        writes = [None] * n_chunks
        gathers[0] = start_gather(0)
        for g in range(n_chunks):
            gathers[g].wait()
            if g + 1 < n_chunks:
                if g + 1 - _NBUF >= 0:
                    writes[g + 1 - _NBUF].wait()
                gathers[g + 1] = start_gather(g + 1)
            writes[g] = pltpu.async_copy(
                rows_v.at[g % _NBUF],
                out_hbm.at[pl.ds(base + g * _CHUNK, _CHUNK)],
                wsems[g % _NBUF],
            )
        for g in range(max(0, n_chunks - _NBUF), n_chunks):
            writes[g].wait()

    return gather_kernel


def kernel(tokens, W_E):
    B = tokens.size
    V, D = W_E.shape
    idx = tokens.reshape(B).astype(jnp.int32)
    out = _make_gather(B, V, D)(W_E, idx)
    return out.reshape(*tokens.shape, D)


# speedup vs baseline: 1.6539x; 1.0491x over previous
import functools

import jax
import jax.numpy as jnp
from jax import lax
from jax.experimental import pallas as pl
from jax.experimental.pallas import tpu as pltpu
from jax.experimental.pallas import tpu_sc as plsc

_info = plsc.get_sparse_core_info()
_NC, _NS = _info.num_cores, _info.num_subcores
_NW = _NC * _NS

_CHUNK = 16
_NBUF = 6


@functools.lru_cache(maxsize=None)
def _make_gather(B, V, D):
    assert B % (_NW * _CHUNK) == 0
    b_per_w = B // _NW
    n_chunks = b_per_w // _CHUNK
    mesh = plsc.VectorSubcoreMesh(core_axis_name="c", subcore_axis_name="s")

    @functools.partial(
        pl.kernel,
        out_type=jax.ShapeDtypeStruct((B, D), jnp.float32),
        mesh=mesh,
        scratch_types=[
            pltpu.VMEM((b_per_w,), jnp.int32),
            pltpu.VMEM((_NBUF, _CHUNK, D), jnp.float32),
            pltpu.SemaphoreType.DMA,
        ]
        + [pltpu.SemaphoreType.DMA] * _NBUF,
    )
    def gather_kernel(table_hbm, idx_hbm, out_hbm, idx_v, rows_v, gsem, *wsems):
        wid = lax.axis_index("s") * _NC + lax.axis_index("c")
        base = wid * b_per_w
        pltpu.sync_copy(idx_hbm.at[pl.ds(base, b_per_w)], idx_v)

        def start_gather(g):
            return pltpu.async_copy(
                table_hbm.at[idx_v.at[pl.ds(g * _CHUNK, _CHUNK)]],
                rows_v.at[g % _NBUF],
                gsem,
            )

        gdepth = _NBUF - 2
        wdepth = 2
        gathers = [None] * n_chunks
        writes = [None] * n_chunks
        for g in range(min(gdepth, n_chunks)):
            gathers[g] = start_gather(g)
        for g in range(n_chunks):
            gathers[g].wait()
            if g + gdepth < n_chunks:
                if g - wdepth >= 0:
                    writes[g - wdepth].wait()
                gathers[g + gdepth] = start_gather(g + gdepth)
            writes[g] = pltpu.async_copy(
                rows_v.at[g % _NBUF],
                out_hbm.at[pl.ds(base + g * _CHUNK, _CHUNK)],
                wsems[g % _NBUF],
            )
        for g in range(max(0, n_chunks - _NBUF), n_chunks):
            writes[g].wait()

    return gather_kernel


def kernel(tokens, W_E):
    B = tokens.size
    V, D = W_E.shape
    idx = tokens.reshape(B).astype(jnp.int32)
    out = _make_gather(B, V, D)(W_E, idx)
    return out.reshape(*tokens.shape, D)


# speedup vs baseline: 1.6558x; 1.0011x over previous
import functools

import jax
import jax.numpy as jnp
from jax import lax
from jax.experimental import pallas as pl
from jax.experimental.pallas import tpu as pltpu
from jax.experimental.pallas import tpu_sc as plsc

_info = plsc.get_sparse_core_info()
_NC, _NS = _info.num_cores, _info.num_subcores
_NW = _NC * _NS

_CHUNK = 16
_NBUF = 7


@functools.lru_cache(maxsize=None)
def _make_gather(B, V, D):
    assert B % (_NW * _CHUNK) == 0
    b_per_w = B // _NW
    n_chunks = b_per_w // _CHUNK
    mesh = plsc.VectorSubcoreMesh(core_axis_name="c", subcore_axis_name="s")

    @functools.partial(
        pl.kernel,
        out_type=jax.ShapeDtypeStruct((B, D), jnp.float32),
        mesh=mesh,
        scratch_types=[
            pltpu.VMEM((b_per_w,), jnp.int32),
            pltpu.VMEM((_NBUF, _CHUNK, D), jnp.float32),
            pltpu.SemaphoreType.DMA,
        ]
        + [pltpu.SemaphoreType.DMA] * _NBUF,
    )
    def gather_kernel(table_hbm, idx_hbm, out_hbm, idx_v, rows_v, gsem, *wsems):
        wid = lax.axis_index("s") * _NC + lax.axis_index("c")
        base = wid * b_per_w
        pltpu.sync_copy(idx_hbm.at[pl.ds(base, b_per_w)], idx_v)

        def start_gather(g):
            return pltpu.async_copy(
                table_hbm.at[idx_v.at[pl.ds(g * _CHUNK, _CHUNK)]],
                rows_v.at[g % _NBUF],
                gsem,
            )

        gdepth = _NBUF - 2
        wdepth = 2
        gathers = [None] * n_chunks
        writes = [None] * n_chunks
        for g in range(min(gdepth, n_chunks)):
            gathers[g] = start_gather(g)
        for g in range(n_chunks):
            gathers[g].wait()
            if g + gdepth < n_chunks:
                if g - wdepth >= 0:
                    writes[g - wdepth].wait()
                gathers[g + gdepth] = start_gather(g + gdepth)
            writes[g] = pltpu.async_copy(
                rows_v.at[g % _NBUF],
                out_hbm.at[pl.ds(base + g * _CHUNK, _CHUNK)],
                wsems[g % _NBUF],
            )
        for g in range(max(0, n_chunks - _NBUF), n_chunks):
            writes[g].wait()

    return gather_kernel


def kernel(tokens, W_E):
    B = tokens.size
    V, D = W_E.shape
    idx = tokens.reshape(B).astype(jnp.int32)
    out = _make_gather(B, V, D)(W_E, idx)
    return out.reshape(*tokens.shape, D)
